# baseline (speedup 1.0000x reference)
import jax
import jax.numpy as jnp
from jax import lax
from jax.experimental import pallas as pl
from jax.experimental.pallas import tpu as pltpu
from jax.experimental.pallas import tpu_sc as plsc

BATCH = 16384
EMBED_DIM = 2
NUM_WORKERS = 16
BPW = BATCH // NUM_WORKERS
OPW = BPW * EMBED_DIM
L = 16
NCHUNK = 4
CHUNK = BPW // NCHUNK


def _sc_body(idx_hbm, tab_hbm, out_hbm, idx_v, tab_v, out_v, sems):
    s = lax.axis_index("s")
    base = s * BPW
    sem_t, sem_i, sem_o = sems
    cp_tab = pltpu.async_copy(tab_hbm, tab_v, sem_t)
    cp_in = [
        pltpu.async_copy(
            idx_hbm.at[pl.ds(base + q * CHUNK, CHUNK)],
            idx_v.at[pl.ds(q * CHUNK, CHUNK)],
            sem_i[q],
        )
        for q in range(NCHUNK)
    ]

    iota = lax.iota(jnp.int32, L)
    two_iota = iota * 2
    cp_tab.wait()
    zero = jnp.zeros((L,), jnp.int32)
    one = jnp.ones((L,), jnp.int32)
    two = jnp.full((L,), 2, jnp.int32)
    t00 = plsc.load_gather(tab_v, [zero, zero])
    t01 = plsc.load_gather(tab_v, [zero, one])
    t10 = plsc.load_gather(tab_v, [one, zero])
    t11 = plsc.load_gather(tab_v, [one, one])
    t20 = plsc.load_gather(tab_v, [two, zero])
    t21 = plsc.load_gather(tab_v, [two, one])

    cp_out = []
    for q in range(NCHUNK):
        cp_in[q].wait()
        for k in range(q * (CHUNK // L), (q + 1) * (CHUNK // L)):
            idx16 = idx_v[pl.ds(k * L, L)]
            g0 = plsc.load_gather(tab_v, [idx16, zero])
            g1 = plsc.load_gather(tab_v, [idx16, one])
            plsc.store_scatter(out_v, [two_iota + k * 2 * L], g0)
            plsc.store_scatter(out_v, [two_iota + (k * 2 * L + 1)], g1)
        cp_out.append(
            pltpu.async_copy(
                out_v.at[pl.ds(q * CHUNK * EMBED_DIM, CHUNK * EMBED_DIM)],
                out_hbm.at[
                    pl.ds(
                        base * EMBED_DIM + q * CHUNK * EMBED_DIM,
                        CHUNK * EMBED_DIM,
                    )
                ],
                sem_o[q],
            )
        )
    for cp in cp_out:
        cp.wait()


def kernel(x, emb_table):
    xi = x.astype(jnp.int32)
    mesh = plsc.VectorSubcoreMesh(
        core_axis_name="c", subcore_axis_name="s", num_cores=1
    )
    out_flat = pl.kernel(
        _sc_body,
        out_type=jax.ShapeDtypeStruct((BATCH * EMBED_DIM,), jnp.float32),
        mesh=mesh,
        compiler_params=pltpu.CompilerParams(needs_layout_passes=False),
        scratch_types=[
            pltpu.VMEM((BPW,), jnp.int32),
            pltpu.VMEM((3, EMBED_DIM), jnp.float32),
            pltpu.VMEM((OPW,), jnp.float32),
            (
                pltpu.SemaphoreType.DMA,
                [pltpu.SemaphoreType.DMA] * NCHUNK,
                [pltpu.SemaphoreType.DMA] * NCHUNK,
            ),
        ],
    )(xi, emb_table)
    return out_flat.reshape(BATCH, EMBED_DIM)
